# baseline (speedup 1.0000x reference)
import functools

import jax
import jax.numpy as jnp
from jax import lax
from jax.experimental import pallas as pl
from jax.experimental.pallas import tpu as pltpu
from jax.experimental.pallas import tpu_sc as plsc

_INFO = plsc.get_sparse_core_info()
_NC = _INFO.num_cores
_NS = _INFO.num_subcores
_NL = _INFO.num_lanes
_NW = _NC * _NS

_BATCH = 16384
_DIM = 64
_BPW = _BATCH // _NW
_CHUNK = 128
_NCHUNK = _BPW // _CHUNK


def _rsqrt(x):
    xi = plsc.bitcast(x, jnp.int32)
    y = plsc.bitcast(jnp.int32(0x5F3759DF) - (xi >> 1), jnp.float32)
    for _ in range(3):
        y = y * (1.5 - 0.5 * x * y * y)
    return y


def _row4(ref, i):
    a = plsc.unpack(ref[i, pl.ds(0, 32)], format=plsc.PackFormat.INTERLEAVED)
    b = plsc.unpack(ref[i, pl.ds(32, 32)], format=plsc.PackFormat.INTERLEAVED)
    return a[0], a[1], b[0], b[1]


def _body(bh, bt, br, ent, rel, out, idx_h, idx_t, idx_r,
          h_rows, t_rows, r_rows, out_v, sem_i, s0, s1, s2, s3):
    wid = lax.axis_index("s") * _NC + lax.axis_index("c")
    sems = (s0, s1, s2, s3)

    cbase = wid * _NCHUNK
    ci = [pltpu.async_copy(src.at[pl.ds(cbase, _NCHUNK)], dst, sem_i)
          for src, dst in ((bh, idx_h), (bt, idx_t), (br, idx_r))]
    for cp in ci:
        cp.wait()

    cps = []
    for c in range(_NCHUNK):
        rows = pl.ds(c * _CHUNK, _CHUNK)
        cps.append([
            pltpu.async_copy(ent.at[idx_h.at[c]], h_rows.at[rows], sems[c]),
            pltpu.async_copy(ent.at[idx_t.at[c]], t_rows.at[rows], sems[c]),
            pltpu.async_copy(rel.at[idx_r.at[c]], r_rows.at[rows], sems[c]),
        ])

    zf = jnp.zeros((_NL,), jnp.float32)
    lane = lax.iota(jnp.int32, _NL)

    for c in range(_NCHUNK):
        for cp in cps[c]:
            cp.wait()

        def group(gi, _, c=c):
            row0 = c * _CHUNK + gi * _NL
            acc = zf
            for r in range(_NL):
                i = row0 + r
                h0, h1, h2, h3 = _row4(h_rows, i)
                t0, t1, t2, t3 = _row4(t_rows, i)
                r0, r1, r2, r3 = _row4(r_rows, i)
                hs = jnp.sum(h0 * h0 + h1 * h1 + h2 * h2 + h3 * h3)
                ts = jnp.sum(t0 * t0 + t1 * t1 + t2 * t2 + t3 * t3)
                rs = jnp.sum(r0 * r0 + r1 * r1 + r2 * r2 + r3 * r3)
                ih = _rsqrt(jnp.maximum(zf + hs, 1e-24))
                it = _rsqrt(jnp.maximum(zf + ts, 1e-24))
                ir = _rsqrt(jnp.maximum(zf + rs, 1e-24))
                sv = (jnp.abs(h0 * ih + r0 * ir - t0 * it)
                      + jnp.abs(h1 * ih + r1 * ir - t1 * it)
                      + jnp.abs(h2 * ih + r2 * ir - t2 * it)
                      + jnp.abs(h3 * ih + r3 * ir - t3 * it))
                s = jnp.sum(sv)
                acc = jnp.where(lane == r, s, acc)
            out_v[pl.ds(row0, _NL)] = acc
            return 0

        lax.fori_loop(0, _CHUNK // _NL, group, 0)

    pltpu.sync_copy(out_v, out.at[pl.ds(wid * _BPW, _BPW)])


def kernel(batch_h, batch_t, batch_r, ent_emb, rel_emb):
    mesh = plsc.VectorSubcoreMesh(core_axis_name="c", subcore_axis_name="s")
    f = functools.partial(
        pl.kernel,
        mesh=mesh,
        compiler_params=pltpu.CompilerParams(
            needs_layout_passes=False, use_tc_tiling_on_sc=False),
        out_type=jax.ShapeDtypeStruct((_BATCH,), jnp.float32),
        scratch_types=[
            pltpu.VMEM((_NCHUNK, _CHUNK), jnp.int32),
            pltpu.VMEM((_NCHUNK, _CHUNK), jnp.int32),
            pltpu.VMEM((_NCHUNK, _CHUNK), jnp.int32),
            pltpu.VMEM((_BPW, _DIM), jnp.bfloat16),
            pltpu.VMEM((_BPW, _DIM), jnp.bfloat16),
            pltpu.VMEM((_BPW, _DIM), jnp.bfloat16),
            pltpu.VMEM((_BPW,), jnp.float32),
            pltpu.SemaphoreType.DMA,
            pltpu.SemaphoreType.DMA,
            pltpu.SemaphoreType.DMA,
            pltpu.SemaphoreType.DMA,
            pltpu.SemaphoreType.DMA,
        ],
    )(_body)
    shape2 = (_NW * _NCHUNK, _CHUNK)
    return f(batch_h.reshape(shape2), batch_t.reshape(shape2),
             batch_r.reshape(shape2),
             ent_emb.astype(jnp.bfloat16), rel_emb.astype(jnp.bfloat16))
